# baseline (speedup 1.0000x reference)
import dataclasses
import functools

import jax
import jax.numpy as jnp
from jax import lax
from jax.experimental import pallas as pl
from jax.experimental.pallas import tpu as pltpu
from jax.experimental.pallas import tpu_sc as plsc
N = 10000
E = 160000
FEAT = 256
D_MODEL = 256
DEGREE = 256

NC = 2
NS = 16
NW = NC * NS
LANES = 16
HR = 79
BN = 10000


def _sc_bincount(edge_local):
    e_local = edge_local.shape[1]
    ntiles = e_local // 128
    tpw = ntiles // NW
    rem = ntiles - tpw * NW

    mesh = plsc.VectorSubcoreMesh(core_axis_name="c", subcore_axis_name="s")
    cp = pltpu.CompilerParams()
    if "needs_layout_passes" in pltpu.CompilerParams.__dataclass_fields__:
        cp = dataclasses.replace(cp, needs_layout_passes=False)

    @functools.partial(
        pl.kernel,
        mesh=mesh,
        compiler_params=cp,
        out_type=jax.ShapeDtypeStruct((NW, HR, 128), jnp.int32),
        scratch_types=[
            pltpu.VMEM((tpw + 1, 2, 128), jnp.int32),
            pltpu.VMEM((HR, 128), jnp.int32),
            pltpu.SemaphoreType.DMA,
        ],
    )
    def bincount_kernel(edge_hbm, out_hbm, tiles_v, hist_v, sem):
        wid = lax.axis_index("s") * NC + lax.axis_index("c")
        t0 = wid * tpw
        zeros16 = jnp.zeros((LANES,), jnp.int32)
        ones16 = jnp.ones((LANES,), jnp.int32)

        copies = [
            pltpu.async_copy(
                edge_hbm.at[:, pl.ds((t0 + k) * 128, 128)], tiles_v.at[k], sem)
            for k in range(tpw)
        ]
        extra = wid < rem
        extra_cp = pltpu.make_async_copy(
            edge_hbm.at[:, pl.ds((NW * tpw + jnp.minimum(wid, rem - 1)) * 128,
                                 128)],
            tiles_v.at[tpw], sem)

        @pl.when(extra)
        def _():
            extra_cp.start()

        @pl.loop(0, HR)
        def _(r):
            @pl.loop(0, 128 // LANES)
            def _(c):
                hist_v[r, pl.ds(c * LANES, LANES)] = zeros16

        for c in copies:
            c.wait()

        @pl.loop(0, tpw * 8)
        def _(i):
            v = tiles_v[i // 8, 1, pl.ds((i % 8) * LANES, LANES)]
            plsc.addupdate_scatter(hist_v, [v >> 7, v & 127], ones16)

        @pl.when(extra)
        def _():
            extra_cp.wait()

            @pl.loop(0, 8)
            def _(j):
                v = tiles_v[tpw, 1, pl.ds(j * LANES, LANES)]
                plsc.addupdate_scatter(hist_v, [v >> 7, v & 127], ones16)

        pltpu.sync_copy(hist_v, out_hbm.at[wid])

    return bincount_kernel(edge_local)


def _tc_body(x_ref, deg_ref, w_ref, b_ref, t_ref, o_ref):
    deg = deg_ref[0, 0]
    iota_d = lax.broadcasted_iota(jnp.int32, (DEGREE, BN), 0)
    onehot_t = (iota_d == deg[None, :]).astype(jnp.float32)
    add = lax.dot_general(onehot_t, t_ref[...], (((0,), (0,)), ((), ())),
                          preferred_element_type=jnp.float32)
    node = lax.dot_general(x_ref[...], w_ref[...], (((1,), (1,)), ((), ())),
                           preferred_element_type=jnp.float32)
    o_ref[...] = node + add + b_ref[...]


def _tc_combine(x, deg3, W, b2, deg_table):
    nb = x.shape[0] // BN
    return pl.pallas_call(
        _tc_body,
        grid=(nb,),
        in_specs=[
            pl.BlockSpec((BN, FEAT), lambda i: (i, 0)),
            pl.BlockSpec((1, 1, BN), lambda i: (i, 0, 0)),
            pl.BlockSpec((D_MODEL, FEAT), lambda i: (0, 0)),
            pl.BlockSpec((1, D_MODEL), lambda i: (0, 0)),
            pl.BlockSpec((DEGREE, D_MODEL), lambda i: (0, 0)),
        ],
        out_specs=pl.BlockSpec((BN, D_MODEL), lambda i: (i, 0)),
        out_shape=jax.ShapeDtypeStruct((x.shape[0], D_MODEL), jnp.float32),
    )(x, deg3, W, b2, deg_table)


def _full_deg3(hist, n_local):
    deg = jnp.minimum(hist.sum(axis=0), DEGREE - 1)
    return deg.reshape(HR * 128)[:n_local].reshape(-1, 1, BN)


def kernel(x, edge_index, W, b, deg_table):
    b2 = b.reshape(1, D_MODEL)
    hist = _sc_bincount(edge_index)
    deg3 = _full_deg3(hist, N)
    return _tc_combine(x, deg3, W, b2, deg_table)


# speedup vs baseline: 1.0689x; 1.0689x over previous
import dataclasses
import functools

import jax
import jax.numpy as jnp
from jax import lax
from jax.experimental import pallas as pl
from jax.experimental.pallas import tpu as pltpu
from jax.experimental.pallas import tpu_sc as plsc
N = 10000
E = 160000
FEAT = 256
D_MODEL = 256
DEGREE = 256

NC = 2
NS = 16
NW = NC * NS
LANES = 16
HR = 79
BN = 5000


def _sc_bincount(edge_local):
    e_local = edge_local.shape[1]
    ntiles = e_local // 128
    tpw = ntiles // NW
    rem = ntiles - tpw * NW

    mesh = plsc.VectorSubcoreMesh(core_axis_name="c", subcore_axis_name="s")
    cp = pltpu.CompilerParams()
    if "needs_layout_passes" in pltpu.CompilerParams.__dataclass_fields__:
        cp = dataclasses.replace(cp, needs_layout_passes=False)

    @functools.partial(
        pl.kernel,
        mesh=mesh,
        compiler_params=cp,
        out_type=jax.ShapeDtypeStruct((NW, HR, 128), jnp.int32),
        scratch_types=[
            pltpu.VMEM((tpw + 1, 2, 128), jnp.int32),
            pltpu.VMEM((HR, 128), jnp.int32),
            pltpu.SemaphoreType.DMA,
        ],
    )
    def bincount_kernel(edge_hbm, out_hbm, tiles_v, hist_v, sem):
        wid = lax.axis_index("s") * NC + lax.axis_index("c")
        t0 = wid * tpw
        zeros16 = jnp.zeros((LANES,), jnp.int32)
        ones16 = jnp.ones((LANES,), jnp.int32)

        copies = [
            pltpu.async_copy(
                edge_hbm.at[:, pl.ds((t0 + k) * 128, 128)], tiles_v.at[k], sem)
            for k in range(tpw)
        ]
        extra = wid < rem
        extra_cp = pltpu.make_async_copy(
            edge_hbm.at[:, pl.ds((NW * tpw + jnp.minimum(wid, rem - 1)) * 128,
                                 128)],
            tiles_v.at[tpw], sem)

        @pl.when(extra)
        def _():
            extra_cp.start()

        @pl.loop(0, HR)
        def _(r):
            @pl.loop(0, 128 // LANES)
            def _(c):
                hist_v[r, pl.ds(c * LANES, LANES)] = zeros16

        for c in copies:
            c.wait()

        @pl.loop(0, tpw * 8)
        def _(i):
            v = tiles_v[i // 8, 1, pl.ds((i % 8) * LANES, LANES)]
            plsc.addupdate_scatter(hist_v, [v >> 7, v & 127], ones16)

        @pl.when(extra)
        def _():
            extra_cp.wait()

            @pl.loop(0, 8)
            def _(j):
                v = tiles_v[tpw, 1, pl.ds(j * LANES, LANES)]
                plsc.addupdate_scatter(hist_v, [v >> 7, v & 127], ones16)

        pltpu.sync_copy(hist_v, out_hbm.at[wid])

    return bincount_kernel(edge_local)


def _tc_body(x_ref, deg_ref, w_ref, b_ref, t_ref, o_ref):
    deg = deg_ref[0, 0]
    iota_d = lax.broadcasted_iota(jnp.int32, (DEGREE, BN), 0)
    onehot_t = (iota_d == deg[None, :]).astype(jnp.float32)
    add = lax.dot_general(onehot_t, t_ref[...], (((0,), (0,)), ((), ())),
                          preferred_element_type=jnp.float32)
    node = lax.dot_general(x_ref[...], w_ref[...], (((1,), (1,)), ((), ())),
                           preferred_element_type=jnp.float32)
    o_ref[...] = node + add + b_ref[...]


def _tc_combine(x, deg3, W, b2, deg_table):
    nb = x.shape[0] // BN
    return pl.pallas_call(
        _tc_body,
        grid=(nb,),
        in_specs=[
            pl.BlockSpec((BN, FEAT), lambda i: (i, 0)),
            pl.BlockSpec((1, 1, BN), lambda i: (i, 0, 0)),
            pl.BlockSpec((D_MODEL, FEAT), lambda i: (0, 0)),
            pl.BlockSpec((1, D_MODEL), lambda i: (0, 0)),
            pl.BlockSpec((DEGREE, D_MODEL), lambda i: (0, 0)),
        ],
        out_specs=pl.BlockSpec((BN, D_MODEL), lambda i: (i, 0)),
        out_shape=jax.ShapeDtypeStruct((x.shape[0], D_MODEL), jnp.float32),
    )(x, deg3, W, b2, deg_table)


def _full_deg3(hist, n_local):
    deg = jnp.minimum(hist.sum(axis=0), DEGREE - 1)
    return deg.reshape(HR * 128)[:n_local].reshape(-1, 1, BN)


def kernel(x, edge_index, W, b, deg_table):
    b2 = b.reshape(1, D_MODEL)
    hist = _sc_bincount(edge_index)
    deg3 = _full_deg3(hist, N)
    return _tc_combine(x, deg3, W, b2, deg_table)


# speedup vs baseline: 1.0730x; 1.0038x over previous
import dataclasses
import functools

import jax
import jax.numpy as jnp
from jax import lax
from jax.experimental import pallas as pl
from jax.experimental.pallas import tpu as pltpu
from jax.experimental.pallas import tpu_sc as plsc
N = 10000
E = 160000
FEAT = 256
D_MODEL = 256
DEGREE = 256

NC = 2
NS = 16
NW = NC * NS
LANES = 16
HR = 79
BN = 5000


def _sc_bincount(edge_local):
    e_local = edge_local.shape[1]
    ntiles = e_local // 128
    tpw = ntiles // NW
    rem = ntiles - tpw * NW

    mesh = plsc.VectorSubcoreMesh(core_axis_name="c", subcore_axis_name="s")
    cp = pltpu.CompilerParams()
    if "needs_layout_passes" in pltpu.CompilerParams.__dataclass_fields__:
        cp = dataclasses.replace(cp, needs_layout_passes=False)

    @functools.partial(
        pl.kernel,
        mesh=mesh,
        compiler_params=cp,
        out_type=jax.ShapeDtypeStruct((NW, HR, 128), jnp.int32),
        scratch_types=[
            pltpu.VMEM((tpw + 1, 2, 128), jnp.int32),
            pltpu.VMEM((HR, 128), jnp.int32),
            pltpu.SemaphoreType.DMA,
        ],
    )
    def bincount_kernel(edge_hbm, out_hbm, tiles_v, hist_v, sem):
        wid = lax.axis_index("s") * NC + lax.axis_index("c")
        t0 = wid * tpw
        zeros16 = jnp.zeros((LANES,), jnp.int32)
        ones16 = jnp.ones((LANES,), jnp.int32)

        copies = [
            pltpu.async_copy(
                edge_hbm.at[:, pl.ds((t0 + k) * 128, 128)], tiles_v.at[k], sem)
            for k in range(tpw)
        ]
        extra = wid < rem
        extra_cp = pltpu.make_async_copy(
            edge_hbm.at[:, pl.ds((NW * tpw + jnp.minimum(wid, rem - 1)) * 128,
                                 128)],
            tiles_v.at[tpw], sem)

        @pl.when(extra)
        def _():
            extra_cp.start()

        @pl.loop(0, HR)
        def _(r):
            for c in range(128 // LANES):
                hist_v[r, pl.ds(c * LANES, LANES)] = zeros16

        for c in copies:
            c.wait()

        @pl.loop(0, tpw)
        def _(k):
            for j in range(8):
                v = tiles_v[k, 1, pl.ds(j * LANES, LANES)]
                plsc.addupdate_scatter(hist_v, [v >> 7, v & 127], ones16)

        @pl.when(extra)
        def _():
            extra_cp.wait()
            for j in range(8):
                v = tiles_v[tpw, 1, pl.ds(j * LANES, LANES)]
                plsc.addupdate_scatter(hist_v, [v >> 7, v & 127], ones16)

        pltpu.sync_copy(hist_v, out_hbm.at[wid])

    return bincount_kernel(edge_local)


def _tc_body(x_ref, deg_ref, w_ref, b_ref, t_ref, o_ref):
    deg = deg_ref[0, 0]
    iota_d = lax.broadcasted_iota(jnp.int32, (DEGREE, BN), 0)
    onehot_t = (iota_d == deg[None, :]).astype(jnp.float32)
    add = lax.dot_general(onehot_t, t_ref[...], (((0,), (0,)), ((), ())),
                          preferred_element_type=jnp.float32)
    node = lax.dot_general(x_ref[...], w_ref[...], (((1,), (1,)), ((), ())),
                           preferred_element_type=jnp.float32)
    o_ref[...] = node + add + b_ref[...]


def _tc_combine(x, deg3, W, b2, deg_table):
    nb = x.shape[0] // BN
    return pl.pallas_call(
        _tc_body,
        grid=(nb,),
        in_specs=[
            pl.BlockSpec((BN, FEAT), lambda i: (i, 0)),
            pl.BlockSpec((1, 1, BN), lambda i: (i, 0, 0)),
            pl.BlockSpec((D_MODEL, FEAT), lambda i: (0, 0)),
            pl.BlockSpec((1, D_MODEL), lambda i: (0, 0)),
            pl.BlockSpec((DEGREE, D_MODEL), lambda i: (0, 0)),
        ],
        out_specs=pl.BlockSpec((BN, D_MODEL), lambda i: (i, 0)),
        out_shape=jax.ShapeDtypeStruct((x.shape[0], D_MODEL), jnp.float32),
    )(x, deg3, W, b2, deg_table)


def _full_deg3(hist, n_local):
    deg = jnp.minimum(hist.sum(axis=0), DEGREE - 1)
    return deg.reshape(HR * 128)[:n_local].reshape(-1, 1, BN)


def kernel(x, edge_index, W, b, deg_table):
    b2 = b.reshape(1, D_MODEL)
    hist = _sc_bincount(edge_index)
    deg3 = _full_deg3(hist, N)
    return _tc_combine(x, deg3, W, b2, deg_table)


# speedup vs baseline: 1.0739x; 1.0008x over previous
import dataclasses
import functools

import jax
import jax.numpy as jnp
from jax import lax
from jax.experimental import pallas as pl
from jax.experimental.pallas import tpu as pltpu
from jax.experimental.pallas import tpu_sc as plsc

N = 10000
E = 160000
FEAT = 256
D_MODEL = 256
DEGREE = 256

NC = 2
NS = 16
NW = NC * NS
LANES = 16
HR = 80
BN = 5000


def _sc_bincount(edge_index):
    ntiles = edge_index.shape[1] // 128
    tpw = ntiles // NW
    rem = ntiles - tpw * NW

    mesh = plsc.VectorSubcoreMesh(core_axis_name="c", subcore_axis_name="s")
    cp = pltpu.CompilerParams()
    if "needs_layout_passes" in pltpu.CompilerParams.__dataclass_fields__:
        cp = dataclasses.replace(cp, needs_layout_passes=False)

    @functools.partial(
        pl.kernel,
        mesh=mesh,
        compiler_params=cp,
        out_type=jax.ShapeDtypeStruct((NC, HR, 128), jnp.int32),
        scratch_types=[
            pltpu.VMEM((tpw + 1, 2, 128), jnp.int32),
            pltpu.VMEM((HR, 128), jnp.int32),
            pltpu.VMEM((HR,), jnp.int32),
            pltpu.VMEM_SHARED((HR, 128), jnp.int32),
            pltpu.SemaphoreType.DMA,
        ],
    )
    def bincount_kernel(edge_hbm, out_hbm, tiles_v, hist_v, rowidx_v, acc_sh,
                        sem):
        cid = lax.axis_index("c")
        sid = lax.axis_index("s")
        wid = sid * NC + cid
        t0 = wid * tpw
        zeros16 = jnp.zeros((LANES,), jnp.int32)
        ones16 = jnp.ones((LANES,), jnp.int32)
        lane = lax.iota(jnp.int32, LANES)

        copies = [
            pltpu.async_copy(
                edge_hbm.at[:, pl.ds((t0 + k) * 128, 128)], tiles_v.at[k], sem)
            for k in range(tpw)
        ]
        extra = wid < rem
        extra_cp = pltpu.make_async_copy(
            edge_hbm.at[:, pl.ds((NW * tpw + jnp.minimum(wid, rem - 1)) * 128,
                                 128)],
            tiles_v.at[tpw], sem)

        @pl.when(extra)
        def _():
            extra_cp.start()

        @pl.loop(0, HR)
        def _(r):
            for c in range(128 // LANES):
                hist_v[r, pl.ds(c * LANES, LANES)] = zeros16

        for c in range(HR // LANES):
            rowidx_v[pl.ds(c * LANES, LANES)] = lane + (c * LANES)

        @pl.when(sid == 0)
        def _():
            pltpu.sync_copy(hist_v, acc_sh)

        plsc.subcore_barrier()

        for c in copies:
            c.wait()

        @pl.loop(0, tpw)
        def _(k):
            for j in range(8):
                v = tiles_v[k, 1, pl.ds(j * LANES, LANES)]
                plsc.addupdate_scatter(hist_v, [v >> 7, v & 127], ones16)

        @pl.when(extra)
        def _():
            extra_cp.wait()
            for j in range(8):
                v = tiles_v[tpw, 1, pl.ds(j * LANES, LANES)]
                plsc.addupdate_scatter(hist_v, [v >> 7, v & 127], ones16)

        pltpu.sync_copy(hist_v, acc_sh.at[rowidx_v], add=True)
        plsc.subcore_barrier()

        @pl.when(sid == 0)
        def _():
            pltpu.sync_copy(acc_sh, out_hbm.at[cid])

    return bincount_kernel(edge_index)


def _tc_body(x_ref, deg_ref, w_ref, b_ref, t_ref, o_ref):
    deg = deg_ref[0, 0]
    iota_d = lax.broadcasted_iota(jnp.int32, (DEGREE, BN), 0)
    onehot_t = (iota_d == deg[None, :]).astype(jnp.float32)
    add = lax.dot_general(onehot_t, t_ref[...], (((0,), (0,)), ((), ())),
                          preferred_element_type=jnp.float32)
    node = lax.dot_general(x_ref[...], w_ref[...], (((1,), (1,)), ((), ())),
                           preferred_element_type=jnp.float32)
    o_ref[...] = node + add + b_ref[...]


def _tc_combine(x, deg3, W, b2, deg_table):
    nb = x.shape[0] // BN
    return pl.pallas_call(
        _tc_body,
        grid=(nb,),
        in_specs=[
            pl.BlockSpec((BN, FEAT), lambda i: (i, 0)),
            pl.BlockSpec((1, 1, BN), lambda i: (i, 0, 0)),
            pl.BlockSpec((D_MODEL, FEAT), lambda i: (0, 0)),
            pl.BlockSpec((1, D_MODEL), lambda i: (0, 0)),
            pl.BlockSpec((DEGREE, D_MODEL), lambda i: (0, 0)),
        ],
        out_specs=pl.BlockSpec((BN, D_MODEL), lambda i: (i, 0)),
        out_shape=jax.ShapeDtypeStruct((x.shape[0], D_MODEL), jnp.float32),
    )(x, deg3, W, b2, deg_table)


def kernel(x, edge_index, W, b, deg_table):
    hist = _sc_bincount(edge_index)
    deg = jnp.minimum(hist.sum(axis=0), DEGREE - 1)
    deg3 = deg.reshape(HR * 128)[:N].reshape(-1, 1, BN)
    return _tc_combine(x, deg3, W, b.reshape(1, D_MODEL), deg_table)


# speedup vs baseline: 1.0772x; 1.0031x over previous
import dataclasses
import functools

import jax
import jax.numpy as jnp
from jax import lax
from jax.experimental import pallas as pl
from jax.experimental.pallas import tpu as pltpu
from jax.experimental.pallas import tpu_sc as plsc

N = 10000
E = 160000
FEAT = 256
D_MODEL = 256
DEGREE = 256

NC = 2
NS = 16
NW = NC * NS
LANES = 16
HR = 80
BN = 5000


def _sc_bincount(edge_index):
    ntiles = edge_index.shape[1] // 128
    tpw = ntiles // NW
    rem = ntiles - tpw * NW

    mesh = plsc.VectorSubcoreMesh(core_axis_name="c", subcore_axis_name="s")
    cp = pltpu.CompilerParams()
    if "needs_layout_passes" in pltpu.CompilerParams.__dataclass_fields__:
        cp = dataclasses.replace(cp, needs_layout_passes=False)

    @functools.partial(
        pl.kernel,
        mesh=mesh,
        compiler_params=cp,
        out_type=jax.ShapeDtypeStruct((NC, HR, 128), jnp.int32),
        scratch_types=[
            pltpu.VMEM((tpw + 1, 2, 128), jnp.int32),
            pltpu.VMEM((HR, 128), jnp.int32),
            pltpu.VMEM((HR,), jnp.int32),
            pltpu.VMEM_SHARED((HR, 128), jnp.int32),
            pltpu.SemaphoreType.DMA,
        ],
    )
    def bincount_kernel(edge_hbm, out_hbm, tiles_v, hist_v, rowidx_v, acc_sh,
                        sem):
        cid = lax.axis_index("c")
        sid = lax.axis_index("s")
        wid = sid * NC + cid
        t0 = wid * tpw
        zeros16 = jnp.zeros((LANES,), jnp.int32)
        ones16 = jnp.ones((LANES,), jnp.int32)
        lane = lax.iota(jnp.int32, LANES)

        copies = [
            pltpu.async_copy(
                edge_hbm.at[:, pl.ds((t0 + k) * 128, 128)], tiles_v.at[k], sem)
            for k in range(tpw)
        ]
        extra = wid < rem
        extra_cp = pltpu.make_async_copy(
            edge_hbm.at[:, pl.ds((NW * tpw + jnp.minimum(wid, rem - 1)) * 128,
                                 128)],
            tiles_v.at[tpw], sem)

        @pl.when(extra)
        def _():
            extra_cp.start()

        @pl.loop(0, HR)
        def _(r):
            for c in range(128 // LANES):
                hist_v[r, pl.ds(c * LANES, LANES)] = zeros16

        for c in range(HR // LANES):
            rowidx_v[pl.ds(c * LANES, LANES)] = lane + (c * LANES)

        @pl.when(sid == 0)
        def _():
            pltpu.sync_copy(hist_v, acc_sh)

        plsc.subcore_barrier()

        for c in copies:
            c.wait()

        @pl.loop(0, tpw)
        def _(k):
            for j in range(8):
                v = tiles_v[k, 1, pl.ds(j * LANES, LANES)]
                plsc.addupdate_scatter(hist_v, [v >> 7, v & 127], ones16)

        @pl.when(extra)
        def _():
            extra_cp.wait()
            for j in range(8):
                v = tiles_v[tpw, 1, pl.ds(j * LANES, LANES)]
                plsc.addupdate_scatter(hist_v, [v >> 7, v & 127], ones16)

        pltpu.sync_copy(hist_v, acc_sh.at[rowidx_v], add=True)
        plsc.subcore_barrier()

        @pl.when(sid == 0)
        def _():
            pltpu.sync_copy(acc_sh, out_hbm.at[cid])

    return bincount_kernel(edge_index)


def _tc_body(x_ref, deg_ref, w_ref, b_ref, t_ref, o_ref):
    deg = deg_ref[0, 0]
    iota_d = lax.broadcasted_iota(jnp.int32, (DEGREE, BN), 0)
    onehot_t = (iota_d == deg[None, :]).astype(jnp.float32)
    add = lax.dot_general(onehot_t, t_ref[...], (((0,), (0,)), ((), ())),
                          preferred_element_type=jnp.float32)
    node = lax.dot_general(x_ref[...], w_ref[...], (((1,), (1,)), ((), ())),
                           preferred_element_type=jnp.float32)
    o_ref[...] = node + add + b_ref[...]


def _tc_combine(x, deg3, W, b2, deg_table):
    nb = x.shape[0] // BN
    return pl.pallas_call(
        _tc_body,
        grid=(nb,),
        in_specs=[
            pl.BlockSpec((BN, FEAT), lambda i: (i, 0)),
            pl.BlockSpec((1, 1, BN), lambda i: (i, 0, 0)),
            pl.BlockSpec((D_MODEL, FEAT), lambda i: (0, 0)),
            pl.BlockSpec((1, D_MODEL), lambda i: (0, 0)),
            pl.BlockSpec((DEGREE, D_MODEL), lambda i: (0, 0)),
        ],
        out_specs=pl.BlockSpec((BN, D_MODEL), lambda i: (i, 0)),
        out_shape=jax.ShapeDtypeStruct((x.shape[0], D_MODEL), jnp.float32),
    )(x, deg3, W, b2, deg_table)


def kernel(x, edge_index, W, b, deg_table):
    hist = _sc_bincount(edge_index)
    deg = jnp.minimum(hist[0] + hist[1], DEGREE - 1)
    deg3 = deg.reshape(HR * 128)[:N].reshape(-1, 1, BN)
    return _tc_combine(x, deg3, W, b.reshape(1, D_MODEL), deg_table)
